# Initial kernel scaffold; baseline (speedup 1.0000x reference)
import jax
import jax.numpy as jnp
from jax.experimental import pallas as pl


def kernel(x, w1, w2, need_backprop):
    raise NotImplementedError("write your pallas kernel here")


# speedup vs baseline: 1.5559x; 1.5559x over previous
import jax
import jax.numpy as jnp
from jax.experimental import pallas as pl
from jax.experimental.pallas import tpu as pltpu


def _fused_kernel(lbl_ref, x_ref, w1_ref, w2_ref, feat_ref, lab_ref):
    xb = x_ref[0].astype(jnp.bfloat16)
    h = jnp.dot(w1_ref[...], xb, preferred_element_type=jnp.float32)
    hb = jnp.maximum(h, 0.0).astype(jnp.bfloat16)
    feat_ref[0] = jnp.dot(w2_ref[...], hb, preferred_element_type=jnp.float32)
    b = pl.program_id(0)
    lab_ref[...] = jnp.full(lab_ref.shape, lbl_ref[b], dtype=jnp.int32)


def kernel(x, w1, w2, need_backprop):
    B, C, H, W = x.shape
    hidden = w1.shape[0]
    out_c = w2.shape[0]
    HW = H * W

    x_r = x.reshape(B, C, HW)

    budget_bytes = 24 * 1024 * 1024
    t_cap = budget_bytes // (4 * (2 * C + hidden) + 2 * (C + hidden))
    t_cap = max(128, (t_cap // 128) * 128)
    hw128 = ((HW + 127) // 128) * 128
    tile = min(hw128, t_cap)
    hw_pad = ((HW + tile - 1) // tile) * tile
    if hw_pad != HW:
        x_r = jnp.pad(x_r, ((0, 0), (0, 0), (0, hw_pad - HW)))

    lbl = need_backprop.astype(jnp.float32).astype(jnp.int32)
    w1b = w1.astype(jnp.bfloat16)
    w2b = w2.astype(jnp.bfloat16)

    feat, lab = pl.pallas_call(
        _fused_kernel,
        out_shape=(
            jax.ShapeDtypeStruct((B, out_c, hw_pad), x.dtype),
            jax.ShapeDtypeStruct((B, 1, hw_pad), jnp.int32),
        ),
        grid_spec=pltpu.PrefetchScalarGridSpec(
            num_scalar_prefetch=1,
            grid=(B, hw_pad // tile),
            in_specs=[
                pl.BlockSpec((1, C, tile), lambda b, j, lbl: (b, 0, j)),
                pl.BlockSpec((hidden, C), lambda b, j, lbl: (0, 0)),
                pl.BlockSpec((out_c, hidden), lambda b, j, lbl: (0, 0)),
            ],
            out_specs=(
                pl.BlockSpec((1, out_c, tile), lambda b, j, lbl: (b, 0, j)),
                pl.BlockSpec((1, 1, tile), lambda b, j, lbl: (b, 0, j)),
            ),
        ),
        compiler_params=pltpu.CompilerParams(
            dimension_semantics=("parallel", "parallel")),
    )(lbl, x_r, w1b, w2b)

    feat = feat[:, :, :HW].reshape(B, out_c, H, W)
    label = lab[:, 0, :HW].reshape(B, H, W)
    return feat, label
